# baseline (speedup 1.0000x reference)
import functools

import jax
import jax.numpy as jnp
from jax import lax
from jax.experimental import pallas as pl
from jax.experimental.pallas import tpu as pltpu
from jax.experimental.pallas import tpu_sc as plsc

_NUM_CORES = 2
_NUM_SUBCORES = 16
_NUM_WORKERS = _NUM_CORES * _NUM_SUBCORES

_DPAD = 128
_CHUNK = 80
_BL = 5
_SPLITS = (10, 40)


def _gather_body(table_hbm, ids_hbm, x_hbm, table_sh, idx_v, rows_a, rows_b, gsa, gsb, ssa, ssb):
    sid = lax.axis_index("s")
    wid = sid * _NUM_CORES + lax.axis_index("c")
    n = ids_hbm.shape[0]
    n_per_w = n // _NUM_WORKERS
    base = pl.multiple_of(wid * n_per_w, 8)

    @pl.when(sid == 0)
    def _():
        pltpu.sync_copy(table_hbm, table_sh)

    pltpu.sync_copy(ids_hbm.at[pl.ds(base, n_per_w)], idx_v)
    plsc.subcore_barrier()

    nch = n_per_w // _CHUNK
    nph = nch // 2

    def g_start(c, buf, sem):
        off = pl.multiple_of(c * _CHUNK, 8)
        pltpu.async_copy(table_sh.at[idx_v.at[pl.ds(off, _CHUNK)]], buf, sem)

    def g_wait(buf, sem):
        pltpu.make_async_copy(table_hbm.at[pl.ds(0, _CHUNK)], buf, sem).wait()

    def s_start(c, buf, sem):
        off = pl.multiple_of(base + c * _CHUNK, 8)
        pltpu.async_copy(buf, x_hbm.at[pl.ds(off, _CHUNK)], sem)

    def s_wait(buf, sem):
        pltpu.make_async_copy(buf, x_hbm.at[pl.ds(base, _CHUNK)], sem).wait()

    g_start(0, rows_a, gsa)
    g_wait(rows_a, gsa)
    s_start(0, rows_a, ssa)
    g_start(1, rows_b, gsb)
    g_wait(rows_b, gsb)
    s_start(1, rows_b, ssb)
    s_wait(rows_a, ssa)
    g_start(2, rows_a, gsa)

    def pair_step(i, carry):
        c0 = i * 2
        g_wait(rows_a, gsa)
        s_start(c0, rows_a, ssa)
        s_wait(rows_b, ssb)
        g_start(c0 + 1, rows_b, gsb)
        g_wait(rows_b, gsb)
        s_start(c0 + 1, rows_b, ssb)
        s_wait(rows_a, ssa)
        g_start(c0 + 2, rows_a, gsa)
        return carry

    lax.fori_loop(1, nph - 1, pair_step, 0)

    c0 = (nph - 1) * 2
    g_wait(rows_a, gsa)
    s_start(c0, rows_a, ssa)
    s_wait(rows_b, ssb)
    g_start(c0 + 1, rows_b, gsb)
    g_wait(rows_b, gsb)
    s_start(c0 + 1, rows_b, ssb)
    s_wait(rows_a, ssa)
    s_wait(rows_b, ssb)


def _proj_kernel(x_ref, w_ref, bias_ref, out_ref):
    for j in range(_BL):
        out_ref[j] = (
            lax.dot_general(
                w_ref[...],
                x_ref[j],
                dimension_numbers=(((1,), (1,)), ((), ())),
                preferred_element_type=jnp.float32,
            )
            + bias_ref[...]
        )


def _proj_update_kernel(x_ref, w_ref, bias_ref, prev_ref, out_ref):
    del prev_ref
    _proj_kernel(x_ref, w_ref, bias_ref, out_ref)


def kernel(input_ids, embed_table, proj_w, proj_b):
    b, l = input_ids.shape
    v, d = embed_table.shape
    n = b * l

    table_pad = jnp.pad(embed_table, ((0, 0), (0, _DPAD - d)))
    w_pad = jnp.pad(proj_w, ((0, 0), (0, _DPAD - d)))
    bias = proj_b.reshape(v, 1)
    ids = input_ids.T.reshape(n).astype(jnp.int32)

    def make_gather(n_c):
        return pl.kernel(
            _gather_body,
            out_type=jax.ShapeDtypeStruct((n_c, _DPAD), jnp.float32),
            mesh=plsc.VectorSubcoreMesh(
                core_axis_name="c", subcore_axis_name="s",
                num_cores=_NUM_CORES, num_subcores=_NUM_SUBCORES,
            ),
            scratch_types=[
                pltpu.VMEM_SHARED((v, _DPAD), jnp.float32),
                pltpu.VMEM((n_c // _NUM_WORKERS,), jnp.int32),
                pltpu.VMEM((_CHUNK, _DPAD), jnp.float32),
                pltpu.VMEM((_CHUNK, _DPAD), jnp.float32),
                pltpu.SemaphoreType.DMA,
                pltpu.SemaphoreType.DMA,
                pltpu.SemaphoreType.DMA,
                pltpu.SemaphoreType.DMA,
            ],
        )

    xs = []
    l_off = 0
    for l_c in _SPLITS:
        n_c = l_c * b
        xs.append(
            make_gather(n_c)(
                table_pad, lax.slice(ids, (l_off * b,), ((l_off + l_c) * b,))
            ).reshape(l_c, b, _DPAD)
        )
        l_off += l_c

    common = dict(
        out_shape=jax.ShapeDtypeStruct((l, v, b), jnp.float32),
        compiler_params=pltpu.CompilerParams(
            dimension_semantics=("arbitrary",),
            vmem_limit_bytes=120 * 1024 * 1024,
        ),
    )
    x_spec = pl.BlockSpec((_BL, b, _DPAD), lambda i: (i, 0, 0))
    w_spec = pl.BlockSpec((v, _DPAD), lambda i: (0, 0))
    b_spec = pl.BlockSpec((v, 1), lambda i: (0, 0))

    out_t = pl.pallas_call(
        _proj_kernel,
        grid=(_SPLITS[0] // _BL,),
        in_specs=[x_spec, w_spec, b_spec],
        out_specs=pl.BlockSpec((_BL, v, b), lambda i: (i, 0, 0)),
        **common,
    )(xs[0], w_pad, bias)

    l_off = _SPLITS[0]
    for c in range(1, len(_SPLITS)):
        out_t = pl.pallas_call(
            _proj_update_kernel,
            grid=(_SPLITS[c] // _BL,),
            in_specs=[
                x_spec,
                w_spec,
                b_spec,
                pl.BlockSpec(memory_space=pl.ANY),
            ],
            out_specs=pl.BlockSpec(
                (_BL, v, b),
                functools.partial(lambda off, i: (off + i, 0, 0), l_off // _BL),
            ),
            input_output_aliases={3: 0},
            **common,
        )(xs[c], w_pad, bias, out_t)
        l_off += _SPLITS[c]

    return jnp.transpose(out_t, (2, 0, 1))


# speedup vs baseline: 1.0048x; 1.0048x over previous
import functools

import jax
import jax.numpy as jnp
from jax import lax
from jax.experimental import pallas as pl
from jax.experimental.pallas import tpu as pltpu
from jax.experimental.pallas import tpu_sc as plsc

_NUM_CORES = 2
_NUM_SUBCORES = 16
_NUM_WORKERS = _NUM_CORES * _NUM_SUBCORES

_DPAD = 128
_CHUNK = 80
_BL = 2
_SPLITS = (10, 40)


def _gather_body(table_hbm, ids_hbm, x_hbm, table_sh, idx_v, rows_a, rows_b, gsa, gsb, ssa, ssb):
    sid = lax.axis_index("s")
    wid = sid * _NUM_CORES + lax.axis_index("c")
    n = ids_hbm.shape[0]
    n_per_w = n // _NUM_WORKERS
    base = pl.multiple_of(wid * n_per_w, 8)

    @pl.when(sid == 0)
    def _():
        pltpu.sync_copy(table_hbm, table_sh)

    pltpu.sync_copy(ids_hbm.at[pl.ds(base, n_per_w)], idx_v)
    plsc.subcore_barrier()

    nch = n_per_w // _CHUNK
    nph = nch // 2

    def g_start(c, buf, sem):
        off = pl.multiple_of(c * _CHUNK, 8)
        pltpu.async_copy(table_sh.at[idx_v.at[pl.ds(off, _CHUNK)]], buf, sem)

    def g_wait(buf, sem):
        pltpu.make_async_copy(table_hbm.at[pl.ds(0, _CHUNK)], buf, sem).wait()

    def s_start(c, buf, sem):
        off = pl.multiple_of(base + c * _CHUNK, 8)
        pltpu.async_copy(buf, x_hbm.at[pl.ds(off, _CHUNK)], sem)

    def s_wait(buf, sem):
        pltpu.make_async_copy(buf, x_hbm.at[pl.ds(base, _CHUNK)], sem).wait()

    g_start(0, rows_a, gsa)
    g_wait(rows_a, gsa)
    s_start(0, rows_a, ssa)
    g_start(1, rows_b, gsb)
    g_wait(rows_b, gsb)
    s_start(1, rows_b, ssb)
    s_wait(rows_a, ssa)
    g_start(2, rows_a, gsa)

    def pair_step(i, carry):
        c0 = i * 2
        g_wait(rows_a, gsa)
        s_start(c0, rows_a, ssa)
        s_wait(rows_b, ssb)
        g_start(c0 + 1, rows_b, gsb)
        g_wait(rows_b, gsb)
        s_start(c0 + 1, rows_b, ssb)
        s_wait(rows_a, ssa)
        g_start(c0 + 2, rows_a, gsa)
        return carry

    lax.fori_loop(1, nph - 1, pair_step, 0)

    c0 = (nph - 1) * 2
    g_wait(rows_a, gsa)
    s_start(c0, rows_a, ssa)
    s_wait(rows_b, ssb)
    g_start(c0 + 1, rows_b, gsb)
    g_wait(rows_b, gsb)
    s_start(c0 + 1, rows_b, ssb)
    s_wait(rows_a, ssa)
    s_wait(rows_b, ssb)


def _proj_kernel(x_ref, w_ref, bias_ref, out_ref):
    for j in range(_BL):
        out_ref[j] = (
            lax.dot_general(
                w_ref[...],
                x_ref[j],
                dimension_numbers=(((1,), (1,)), ((), ())),
                preferred_element_type=jnp.float32,
            )
            + bias_ref[...]
        )


def _proj_update_kernel(x_ref, w_ref, bias_ref, prev_ref, out_ref):
    del prev_ref
    _proj_kernel(x_ref, w_ref, bias_ref, out_ref)


def kernel(input_ids, embed_table, proj_w, proj_b):
    b, l = input_ids.shape
    v, d = embed_table.shape
    n = b * l

    table_pad = jnp.pad(embed_table, ((0, 0), (0, _DPAD - d)))
    w_pad = jnp.pad(proj_w, ((0, 0), (0, _DPAD - d)))
    bias = proj_b.reshape(v, 1)
    ids = input_ids.T.reshape(n).astype(jnp.int32)

    def make_gather(n_c):
        return pl.kernel(
            _gather_body,
            out_type=jax.ShapeDtypeStruct((n_c, _DPAD), jnp.float32),
            mesh=plsc.VectorSubcoreMesh(
                core_axis_name="c", subcore_axis_name="s",
                num_cores=_NUM_CORES, num_subcores=_NUM_SUBCORES,
            ),
            scratch_types=[
                pltpu.VMEM_SHARED((v, _DPAD), jnp.float32),
                pltpu.VMEM((n_c // _NUM_WORKERS,), jnp.int32),
                pltpu.VMEM((_CHUNK, _DPAD), jnp.float32),
                pltpu.VMEM((_CHUNK, _DPAD), jnp.float32),
                pltpu.SemaphoreType.DMA,
                pltpu.SemaphoreType.DMA,
                pltpu.SemaphoreType.DMA,
                pltpu.SemaphoreType.DMA,
            ],
        )

    xs = []
    l_off = 0
    for l_c in _SPLITS:
        n_c = l_c * b
        xs.append(
            make_gather(n_c)(
                table_pad, lax.slice(ids, (l_off * b,), ((l_off + l_c) * b,))
            ).reshape(l_c, b, _DPAD)
        )
        l_off += l_c

    common = dict(
        out_shape=jax.ShapeDtypeStruct((l, v, b), jnp.float32),
        compiler_params=pltpu.CompilerParams(
            dimension_semantics=("arbitrary",),
            vmem_limit_bytes=120 * 1024 * 1024,
        ),
    )
    x_spec = pl.BlockSpec((_BL, b, _DPAD), lambda i: (i, 0, 0))
    w_spec = pl.BlockSpec((v, _DPAD), lambda i: (0, 0))
    b_spec = pl.BlockSpec((v, 1), lambda i: (0, 0))

    out_t = pl.pallas_call(
        _proj_kernel,
        grid=(_SPLITS[0] // _BL,),
        in_specs=[x_spec, w_spec, b_spec],
        out_specs=pl.BlockSpec((_BL, v, b), lambda i: (i, 0, 0)),
        **common,
    )(xs[0], w_pad, bias)

    l_off = _SPLITS[0]
    for c in range(1, len(_SPLITS)):
        out_t = pl.pallas_call(
            _proj_update_kernel,
            grid=(_SPLITS[c] // _BL,),
            in_specs=[
                x_spec,
                w_spec,
                b_spec,
                pl.BlockSpec(memory_space=pl.ANY),
            ],
            out_specs=pl.BlockSpec(
                (_BL, v, b),
                functools.partial(lambda off, i: (off + i, 0, 0), l_off // _BL),
            ),
            input_output_aliases={3: 0},
            **common,
        )(xs[c], w_pad, bias, out_t)
        l_off += _SPLITS[c]

    return jnp.transpose(out_t, (2, 0, 1))


# speedup vs baseline: 1.0355x; 1.0305x over previous
import functools

import jax
import jax.numpy as jnp
from jax import lax
from jax.experimental import pallas as pl
from jax.experimental.pallas import tpu as pltpu
from jax.experimental.pallas import tpu_sc as plsc

_NUM_CORES = 2
_NUM_SUBCORES = 16
_NUM_WORKERS = _NUM_CORES * _NUM_SUBCORES

_DPAD = 128
_CHUNK = 80
_BL = 2
_SPLITS = (10, 40)


def _gather_body(table_hbm, ids_hbm, x_hbm, table_sh, idx_v, rows_a, rows_b, gsa, gsb, ssa, ssb):
    sid = lax.axis_index("s")
    wid = sid * _NUM_CORES + lax.axis_index("c")
    n = ids_hbm.shape[0]
    n_per_w = n // _NUM_WORKERS
    base = pl.multiple_of(wid * n_per_w, 8)

    @pl.when(sid == 0)
    def _():
        pltpu.sync_copy(table_hbm, table_sh)

    pltpu.sync_copy(ids_hbm.at[pl.ds(base, n_per_w)], idx_v)
    plsc.subcore_barrier()

    nch = n_per_w // _CHUNK
    nph = nch // 2

    def g_start(c, buf, sem):
        off = pl.multiple_of(c * _CHUNK, 8)
        pltpu.async_copy(table_sh.at[idx_v.at[pl.ds(off, _CHUNK)]], buf, sem)

    def g_wait(buf, sem):
        pltpu.make_async_copy(table_hbm.at[pl.ds(0, _CHUNK)], buf, sem).wait()

    def s_start(c, buf, sem):
        off = pl.multiple_of(base + c * _CHUNK, 8)
        pltpu.async_copy(buf, x_hbm.at[pl.ds(off, _CHUNK)], sem)

    def s_wait(buf, sem):
        pltpu.make_async_copy(buf, x_hbm.at[pl.ds(base, _CHUNK)], sem).wait()

    g_start(0, rows_a, gsa)
    g_wait(rows_a, gsa)
    s_start(0, rows_a, ssa)
    g_start(1, rows_b, gsb)
    g_wait(rows_b, gsb)
    s_start(1, rows_b, ssb)
    s_wait(rows_a, ssa)
    g_start(2, rows_a, gsa)

    def pair_step(i, carry):
        c0 = i * 2
        g_wait(rows_a, gsa)
        s_start(c0, rows_a, ssa)
        s_wait(rows_b, ssb)
        g_start(c0 + 1, rows_b, gsb)
        g_wait(rows_b, gsb)
        s_start(c0 + 1, rows_b, ssb)
        s_wait(rows_a, ssa)
        g_start(c0 + 2, rows_a, gsa)
        return carry

    lax.fori_loop(1, nph - 1, pair_step, 0)

    c0 = (nph - 1) * 2
    g_wait(rows_a, gsa)
    s_start(c0, rows_a, ssa)
    s_wait(rows_b, ssb)
    g_start(c0 + 1, rows_b, gsb)
    g_wait(rows_b, gsb)
    s_start(c0 + 1, rows_b, ssb)
    s_wait(rows_a, ssa)
    s_wait(rows_b, ssb)


def _proj_kernel(x_ref, w_ref, bias_ref, out_ref):
    for j in range(_BL):
        out_ref[j] = (
            lax.dot_general(
                w_ref[...],
                x_ref[j],
                dimension_numbers=(((1,), (1,)), ((), ())),
                preferred_element_type=jnp.float32,
            )
            + bias_ref[...]
        )


def _proj_update_kernel(x_ref, w_ref, bias_ref, prev_ref, out_ref):
    del prev_ref
    _proj_kernel(x_ref, w_ref, bias_ref, out_ref)


def _onehot_proj_kernel(ids_ref, tt_ref, w_ref, bias_ref, out_ref):
    i = pl.program_id(0)
    v, bsz = out_ref.shape[1], out_ref.shape[2]
    for j in range(_BL):
        idrow = ids_ref[pl.ds(i * _BL + j, 1)]
        sel = jnp.where(
            lax.broadcasted_iota(jnp.int32, (v, bsz), 0) == idrow,
            jnp.float32(1),
            jnp.float32(0),
        )
        xt = jnp.dot(tt_ref[...], sel, preferred_element_type=jnp.float32)
        out_ref[j] = (
            jnp.dot(w_ref[...], xt, preferred_element_type=jnp.float32)
            + bias_ref[...]
        )


def kernel(input_ids, embed_table, proj_w, proj_b):
    b, l = input_ids.shape
    v, d = embed_table.shape
    n = b * l

    table_pad = jnp.pad(embed_table, ((0, 0), (0, _DPAD - d)))
    w_pad = jnp.pad(proj_w, ((0, 0), (0, _DPAD - d)))
    bias = proj_b.reshape(v, 1)
    ids = input_ids.T.reshape(n).astype(jnp.int32)

    def make_gather(n_c):
        return pl.kernel(
            _gather_body,
            out_type=jax.ShapeDtypeStruct((n_c, _DPAD), jnp.float32),
            mesh=plsc.VectorSubcoreMesh(
                core_axis_name="c", subcore_axis_name="s",
                num_cores=_NUM_CORES, num_subcores=_NUM_SUBCORES,
            ),
            scratch_types=[
                pltpu.VMEM_SHARED((v, _DPAD), jnp.float32),
                pltpu.VMEM((n_c // _NUM_WORKERS,), jnp.int32),
                pltpu.VMEM((_CHUNK, _DPAD), jnp.float32),
                pltpu.VMEM((_CHUNK, _DPAD), jnp.float32),
                pltpu.SemaphoreType.DMA,
                pltpu.SemaphoreType.DMA,
                pltpu.SemaphoreType.DMA,
                pltpu.SemaphoreType.DMA,
            ],
        )

    xs = [None]
    l_off = _SPLITS[0]
    for l_c in _SPLITS[1:]:
        n_c = l_c * b
        xs.append(
            make_gather(n_c)(
                table_pad, lax.slice(ids, (l_off * b,), ((l_off + l_c) * b,))
            ).reshape(l_c, b, _DPAD)
        )
        l_off += l_c

    common = dict(
        out_shape=jax.ShapeDtypeStruct((l, v, b), jnp.float32),
        compiler_params=pltpu.CompilerParams(
            dimension_semantics=("arbitrary",),
            vmem_limit_bytes=120 * 1024 * 1024,
        ),
    )
    x_spec = pl.BlockSpec((_BL, b, _DPAD), lambda i: (i, 0, 0))
    w_spec = pl.BlockSpec((v, _DPAD), lambda i: (0, 0))
    b_spec = pl.BlockSpec((v, 1), lambda i: (0, 0))

    ids0 = lax.slice(ids, (0,), (_SPLITS[0] * b,)).reshape(_SPLITS[0], b)
    tt_pad = jnp.pad(embed_table.T, ((0, _DPAD - d), (0, 0)))
    out_t = pl.pallas_call(
        _onehot_proj_kernel,
        grid=(_SPLITS[0] // _BL,),
        in_specs=[
            pl.BlockSpec((_SPLITS[0], b), lambda i: (0, 0)),
            pl.BlockSpec((_DPAD, v), lambda i: (0, 0)),
            w_spec,
            b_spec,
        ],
        out_specs=pl.BlockSpec((_BL, v, b), lambda i: (i, 0, 0)),
        **common,
    )(ids0, tt_pad, w_pad, bias)

    l_off = _SPLITS[0]
    for c in range(1, len(_SPLITS)):
        out_t = pl.pallas_call(
            _proj_update_kernel,
            grid=(_SPLITS[c] // _BL,),
            in_specs=[
                x_spec,
                w_spec,
                b_spec,
                pl.BlockSpec(memory_space=pl.ANY),
            ],
            out_specs=pl.BlockSpec(
                (_BL, v, b),
                functools.partial(lambda off, i: (off + i, 0, 0), l_off // _BL),
            ),
            input_output_aliases={3: 0},
            **common,
        )(xs[c], w_pad, bias, out_t)
        l_off += _SPLITS[c]

    return jnp.transpose(out_t, (2, 0, 1))


# speedup vs baseline: 1.0360x; 1.0005x over previous
import functools

import jax
import jax.numpy as jnp
from jax import lax
from jax.experimental import pallas as pl
from jax.experimental.pallas import tpu as pltpu
from jax.experimental.pallas import tpu_sc as plsc

_NUM_CORES = 2
_NUM_SUBCORES = 16
_NUM_WORKERS = _NUM_CORES * _NUM_SUBCORES

_DPAD = 128
_CHUNK = 64
_BL = 2
_SPLITS = (14, 36)


def _gather_body(table_hbm, ids_hbm, x_hbm, table_sh, idx_v, rows_a, rows_b, gsa, gsb, ssa, ssb):
    sid = lax.axis_index("s")
    wid = sid * _NUM_CORES + lax.axis_index("c")
    n = ids_hbm.shape[0]
    n_per_w = n // _NUM_WORKERS
    base = pl.multiple_of(wid * n_per_w, 8)

    @pl.when(sid == 0)
    def _():
        pltpu.sync_copy(table_hbm, table_sh)

    pltpu.sync_copy(ids_hbm.at[pl.ds(base, n_per_w)], idx_v)
    plsc.subcore_barrier()

    nch = n_per_w // _CHUNK
    nph = nch // 2

    def g_start(c, buf, sem):
        off = pl.multiple_of(c * _CHUNK, 8)
        pltpu.async_copy(table_sh.at[idx_v.at[pl.ds(off, _CHUNK)]], buf, sem)

    def g_wait(buf, sem):
        pltpu.make_async_copy(table_hbm.at[pl.ds(0, _CHUNK)], buf, sem).wait()

    def s_start(c, buf, sem):
        off = pl.multiple_of(base + c * _CHUNK, 8)
        pltpu.async_copy(buf, x_hbm.at[pl.ds(off, _CHUNK)], sem)

    def s_wait(buf, sem):
        pltpu.make_async_copy(buf, x_hbm.at[pl.ds(base, _CHUNK)], sem).wait()

    g_start(0, rows_a, gsa)
    g_wait(rows_a, gsa)
    s_start(0, rows_a, ssa)
    g_start(1, rows_b, gsb)
    g_wait(rows_b, gsb)
    s_start(1, rows_b, ssb)
    s_wait(rows_a, ssa)
    g_start(2, rows_a, gsa)

    def pair_step(i, carry):
        c0 = i * 2
        g_wait(rows_a, gsa)
        s_start(c0, rows_a, ssa)
        s_wait(rows_b, ssb)
        g_start(c0 + 1, rows_b, gsb)
        g_wait(rows_b, gsb)
        s_start(c0 + 1, rows_b, ssb)
        s_wait(rows_a, ssa)
        g_start(c0 + 2, rows_a, gsa)
        return carry

    lax.fori_loop(1, nph - 1, pair_step, 0)

    c0 = (nph - 1) * 2
    g_wait(rows_a, gsa)
    s_start(c0, rows_a, ssa)
    s_wait(rows_b, ssb)
    g_start(c0 + 1, rows_b, gsb)
    g_wait(rows_b, gsb)
    s_start(c0 + 1, rows_b, ssb)
    s_wait(rows_a, ssa)
    s_wait(rows_b, ssb)


def _proj_kernel(x_ref, w_ref, bias_ref, out_ref):
    for j in range(_BL):
        out_ref[j] = (
            lax.dot_general(
                w_ref[...],
                x_ref[j],
                dimension_numbers=(((1,), (1,)), ((), ())),
                preferred_element_type=jnp.float32,
            )
            + bias_ref[...]
        )


def _proj_update_kernel(x_ref, w_ref, bias_ref, prev_ref, out_ref):
    del prev_ref
    _proj_kernel(x_ref, w_ref, bias_ref, out_ref)


def _onehot_proj_kernel(ids_ref, tt_ref, w_ref, bias_ref, out_ref):
    i = pl.program_id(0)
    v, bsz = out_ref.shape[1], out_ref.shape[2]
    for j in range(_BL):
        idrow = ids_ref[pl.ds(i * _BL + j, 1)]
        sel = jnp.where(
            lax.broadcasted_iota(jnp.int32, (v, bsz), 0) == idrow,
            jnp.float32(1),
            jnp.float32(0),
        )
        xt = jnp.dot(tt_ref[...], sel, preferred_element_type=jnp.float32)
        out_ref[j] = (
            jnp.dot(w_ref[...], xt, preferred_element_type=jnp.float32)
            + bias_ref[...]
        )


def kernel(input_ids, embed_table, proj_w, proj_b):
    b, l = input_ids.shape
    v, d = embed_table.shape
    n = b * l

    table_pad = jnp.pad(embed_table, ((0, 0), (0, _DPAD - d)))
    w_pad = jnp.pad(proj_w, ((0, 0), (0, _DPAD - d)))
    bias = proj_b.reshape(v, 1)
    ids = input_ids.T.reshape(n).astype(jnp.int32)

    def make_gather(n_c):
        return pl.kernel(
            _gather_body,
            out_type=jax.ShapeDtypeStruct((n_c, _DPAD), jnp.float32),
            mesh=plsc.VectorSubcoreMesh(
                core_axis_name="c", subcore_axis_name="s",
                num_cores=_NUM_CORES, num_subcores=_NUM_SUBCORES,
            ),
            scratch_types=[
                pltpu.VMEM_SHARED((v, _DPAD), jnp.float32),
                pltpu.VMEM((n_c // _NUM_WORKERS,), jnp.int32),
                pltpu.VMEM((_CHUNK, _DPAD), jnp.float32),
                pltpu.VMEM((_CHUNK, _DPAD), jnp.float32),
                pltpu.SemaphoreType.DMA,
                pltpu.SemaphoreType.DMA,
                pltpu.SemaphoreType.DMA,
                pltpu.SemaphoreType.DMA,
            ],
        )

    xs = [None]
    l_off = _SPLITS[0]
    for l_c in _SPLITS[1:]:
        n_c = l_c * b
        xs.append(
            make_gather(n_c)(
                table_pad, lax.slice(ids, (l_off * b,), ((l_off + l_c) * b,))
            ).reshape(l_c, b, _DPAD)
        )
        l_off += l_c

    common = dict(
        out_shape=jax.ShapeDtypeStruct((l, v, b), jnp.float32),
        compiler_params=pltpu.CompilerParams(
            dimension_semantics=("arbitrary",),
            vmem_limit_bytes=120 * 1024 * 1024,
        ),
    )
    x_spec = pl.BlockSpec((_BL, b, _DPAD), lambda i: (i, 0, 0))
    w_spec = pl.BlockSpec((v, _DPAD), lambda i: (0, 0))
    b_spec = pl.BlockSpec((v, 1), lambda i: (0, 0))

    ids0 = lax.slice(ids, (0,), (_SPLITS[0] * b,)).reshape(_SPLITS[0], b)
    tt_pad = jnp.pad(embed_table.T, ((0, _DPAD - d), (0, 0)))
    out_t = pl.pallas_call(
        _onehot_proj_kernel,
        grid=(_SPLITS[0] // _BL,),
        in_specs=[
            pl.BlockSpec((_SPLITS[0], b), lambda i: (0, 0)),
            pl.BlockSpec((_DPAD, v), lambda i: (0, 0)),
            w_spec,
            b_spec,
        ],
        out_specs=pl.BlockSpec((_BL, v, b), lambda i: (i, 0, 0)),
        **common,
    )(ids0, tt_pad, w_pad, bias)

    l_off = _SPLITS[0]
    for c in range(1, len(_SPLITS)):
        out_t = pl.pallas_call(
            _proj_update_kernel,
            grid=(_SPLITS[c] // _BL,),
            in_specs=[
                x_spec,
                w_spec,
                b_spec,
                pl.BlockSpec(memory_space=pl.ANY),
            ],
            out_specs=pl.BlockSpec(
                (_BL, v, b),
                functools.partial(lambda off, i: (off + i, 0, 0), l_off // _BL),
            ),
            input_output_aliases={3: 0},
            **common,
        )(xs[c], w_pad, bias, out_t)
        l_off += _SPLITS[c]

    return jnp.transpose(out_t, (2, 0, 1))


# speedup vs baseline: 1.0456x; 1.0093x over previous
import functools

import jax
import jax.numpy as jnp
from jax import lax
from jax.experimental import pallas as pl
from jax.experimental.pallas import tpu as pltpu
from jax.experimental.pallas import tpu_sc as plsc

_NUM_CORES = 2
_NUM_SUBCORES = 16
_NUM_WORKERS = _NUM_CORES * _NUM_SUBCORES

_DPAD = 128
_CHUNK = 64
_BL = 2
_SPLITS = (18, 32)


def _gather_body(table_hbm, ids_hbm, x_hbm, table_sh, idx_v, rows_a, rows_b, gsa, gsb, ssa, ssb):
    sid = lax.axis_index("s")
    wid = sid * _NUM_CORES + lax.axis_index("c")
    n = ids_hbm.shape[0]
    n_per_w = n // _NUM_WORKERS
    base = pl.multiple_of(wid * n_per_w, 8)

    @pl.when(sid == 0)
    def _():
        pltpu.sync_copy(table_hbm, table_sh)

    pltpu.sync_copy(ids_hbm.at[pl.ds(base, n_per_w)], idx_v)
    plsc.subcore_barrier()

    nch = n_per_w // _CHUNK
    nph = nch // 2

    def g_start(c, buf, sem):
        off = pl.multiple_of(c * _CHUNK, 8)
        pltpu.async_copy(table_sh.at[idx_v.at[pl.ds(off, _CHUNK)]], buf, sem)

    def g_wait(buf, sem):
        pltpu.make_async_copy(table_hbm.at[pl.ds(0, _CHUNK)], buf, sem).wait()

    def s_start(c, buf, sem):
        off = pl.multiple_of(base + c * _CHUNK, 8)
        pltpu.async_copy(buf, x_hbm.at[pl.ds(off, _CHUNK)], sem)

    def s_wait(buf, sem):
        pltpu.make_async_copy(buf, x_hbm.at[pl.ds(base, _CHUNK)], sem).wait()

    g_start(0, rows_a, gsa)
    g_wait(rows_a, gsa)
    s_start(0, rows_a, ssa)
    g_start(1, rows_b, gsb)
    g_wait(rows_b, gsb)
    s_start(1, rows_b, ssb)
    s_wait(rows_a, ssa)
    g_start(2, rows_a, gsa)

    def pair_step(i, carry):
        c0 = i * 2
        g_wait(rows_a, gsa)
        s_start(c0, rows_a, ssa)
        s_wait(rows_b, ssb)
        g_start(c0 + 1, rows_b, gsb)
        g_wait(rows_b, gsb)
        s_start(c0 + 1, rows_b, ssb)
        s_wait(rows_a, ssa)
        g_start(c0 + 2, rows_a, gsa)
        return carry

    lax.fori_loop(1, nph - 1, pair_step, 0)

    c0 = (nph - 1) * 2
    g_wait(rows_a, gsa)
    s_start(c0, rows_a, ssa)
    s_wait(rows_b, ssb)
    g_start(c0 + 1, rows_b, gsb)
    g_wait(rows_b, gsb)
    s_start(c0 + 1, rows_b, ssb)
    s_wait(rows_a, ssa)
    s_wait(rows_b, ssb)


def _proj_kernel(x_ref, w_ref, bias_ref, out_ref):
    for j in range(_BL):
        out_ref[j] = (
            lax.dot_general(
                w_ref[...],
                x_ref[j],
                dimension_numbers=(((1,), (1,)), ((), ())),
                preferred_element_type=jnp.float32,
            )
            + bias_ref[...]
        )


def _proj_update_kernel(x_ref, w_ref, bias_ref, prev_ref, out_ref):
    del prev_ref
    _proj_kernel(x_ref, w_ref, bias_ref, out_ref)


def _onehot_proj_kernel(ids_ref, tt_ref, w_ref, bias_ref, out_ref):
    i = pl.program_id(0)
    v, bsz = out_ref.shape[1], out_ref.shape[2]
    for j in range(_BL):
        idrow = ids_ref[pl.ds(i * _BL + j, 1)]
        sel = jnp.where(
            lax.broadcasted_iota(jnp.int32, (v, bsz), 0) == idrow,
            jnp.float32(1),
            jnp.float32(0),
        )
        xt = jnp.dot(tt_ref[...], sel, preferred_element_type=jnp.float32)
        out_ref[j] = (
            jnp.dot(w_ref[...], xt, preferred_element_type=jnp.float32)
            + bias_ref[...]
        )


def kernel(input_ids, embed_table, proj_w, proj_b):
    b, l = input_ids.shape
    v, d = embed_table.shape
    n = b * l

    table_pad = jnp.pad(embed_table, ((0, 0), (0, _DPAD - d)))
    w_pad = jnp.pad(proj_w, ((0, 0), (0, _DPAD - d)))
    bias = proj_b.reshape(v, 1)
    ids = input_ids.T.reshape(n).astype(jnp.int32)

    def make_gather(n_c):
        return pl.kernel(
            _gather_body,
            out_type=jax.ShapeDtypeStruct((n_c, _DPAD), jnp.float32),
            mesh=plsc.VectorSubcoreMesh(
                core_axis_name="c", subcore_axis_name="s",
                num_cores=_NUM_CORES, num_subcores=_NUM_SUBCORES,
            ),
            scratch_types=[
                pltpu.VMEM_SHARED((v, _DPAD), jnp.float32),
                pltpu.VMEM((n_c // _NUM_WORKERS,), jnp.int32),
                pltpu.VMEM((_CHUNK, _DPAD), jnp.float32),
                pltpu.VMEM((_CHUNK, _DPAD), jnp.float32),
                pltpu.SemaphoreType.DMA,
                pltpu.SemaphoreType.DMA,
                pltpu.SemaphoreType.DMA,
                pltpu.SemaphoreType.DMA,
            ],
        )

    xs = [None]
    l_off = _SPLITS[0]
    for l_c in _SPLITS[1:]:
        n_c = l_c * b
        xs.append(
            make_gather(n_c)(
                table_pad, lax.slice(ids, (l_off * b,), ((l_off + l_c) * b,))
            ).reshape(l_c, b, _DPAD)
        )
        l_off += l_c

    common = dict(
        out_shape=jax.ShapeDtypeStruct((l, v, b), jnp.float32),
        compiler_params=pltpu.CompilerParams(
            dimension_semantics=("arbitrary",),
            vmem_limit_bytes=120 * 1024 * 1024,
        ),
    )
    x_spec = pl.BlockSpec((_BL, b, _DPAD), lambda i: (i, 0, 0))
    w_spec = pl.BlockSpec((v, _DPAD), lambda i: (0, 0))
    b_spec = pl.BlockSpec((v, 1), lambda i: (0, 0))

    ids0 = lax.slice(ids, (0,), (_SPLITS[0] * b,)).reshape(_SPLITS[0], b)
    tt_pad = jnp.pad(embed_table.T, ((0, _DPAD - d), (0, 0)))
    out_t = pl.pallas_call(
        _onehot_proj_kernel,
        grid=(_SPLITS[0] // _BL,),
        in_specs=[
            pl.BlockSpec((_SPLITS[0], b), lambda i: (0, 0)),
            pl.BlockSpec((_DPAD, v), lambda i: (0, 0)),
            w_spec,
            b_spec,
        ],
        out_specs=pl.BlockSpec((_BL, v, b), lambda i: (i, 0, 0)),
        **common,
    )(ids0, tt_pad, w_pad, bias)

    l_off = _SPLITS[0]
    for c in range(1, len(_SPLITS)):
        out_t = pl.pallas_call(
            _proj_update_kernel,
            grid=(_SPLITS[c] // _BL,),
            in_specs=[
                x_spec,
                w_spec,
                b_spec,
                pl.BlockSpec(memory_space=pl.ANY),
            ],
            out_specs=pl.BlockSpec(
                (_BL, v, b),
                functools.partial(lambda off, i: (off + i, 0, 0), l_off // _BL),
            ),
            input_output_aliases={3: 0},
            **common,
        )(xs[c], w_pad, bias, out_t)
        l_off += _SPLITS[c]

    return jnp.transpose(out_t, (2, 0, 1))


# speedup vs baseline: 1.0585x; 1.0123x over previous
import functools

import jax
import jax.numpy as jnp
from jax import lax
from jax.experimental import pallas as pl
from jax.experimental.pallas import tpu as pltpu
from jax.experimental.pallas import tpu_sc as plsc

_NUM_CORES = 2
_NUM_SUBCORES = 16
_NUM_WORKERS = _NUM_CORES * _NUM_SUBCORES

_DPAD = 128
_CHUNK = 64
_BL = 2
_SPLITS = (22, 28)


def _gather_body(table_hbm, ids_hbm, x_hbm, table_sh, idx_v, rows_a, rows_b, gsa, gsb, ssa, ssb):
    sid = lax.axis_index("s")
    wid = sid * _NUM_CORES + lax.axis_index("c")
    n = ids_hbm.shape[0]
    n_per_w = n // _NUM_WORKERS
    base = pl.multiple_of(wid * n_per_w, 8)

    @pl.when(sid == 0)
    def _():
        pltpu.sync_copy(table_hbm, table_sh)

    pltpu.sync_copy(ids_hbm.at[pl.ds(base, n_per_w)], idx_v)
    plsc.subcore_barrier()

    nch = n_per_w // _CHUNK
    nph = nch // 2

    def g_start(c, buf, sem):
        off = pl.multiple_of(c * _CHUNK, 8)
        pltpu.async_copy(table_sh.at[idx_v.at[pl.ds(off, _CHUNK)]], buf, sem)

    def g_wait(buf, sem):
        pltpu.make_async_copy(table_hbm.at[pl.ds(0, _CHUNK)], buf, sem).wait()

    def s_start(c, buf, sem):
        off = pl.multiple_of(base + c * _CHUNK, 8)
        pltpu.async_copy(buf, x_hbm.at[pl.ds(off, _CHUNK)], sem)

    def s_wait(buf, sem):
        pltpu.make_async_copy(buf, x_hbm.at[pl.ds(base, _CHUNK)], sem).wait()

    g_start(0, rows_a, gsa)
    g_wait(rows_a, gsa)
    s_start(0, rows_a, ssa)
    g_start(1, rows_b, gsb)
    g_wait(rows_b, gsb)
    s_start(1, rows_b, ssb)
    s_wait(rows_a, ssa)
    g_start(2, rows_a, gsa)

    def pair_step(i, carry):
        c0 = i * 2
        g_wait(rows_a, gsa)
        s_start(c0, rows_a, ssa)
        s_wait(rows_b, ssb)
        g_start(c0 + 1, rows_b, gsb)
        g_wait(rows_b, gsb)
        s_start(c0 + 1, rows_b, ssb)
        s_wait(rows_a, ssa)
        g_start(c0 + 2, rows_a, gsa)
        return carry

    lax.fori_loop(1, nph - 1, pair_step, 0)

    c0 = (nph - 1) * 2
    g_wait(rows_a, gsa)
    s_start(c0, rows_a, ssa)
    s_wait(rows_b, ssb)
    g_start(c0 + 1, rows_b, gsb)
    g_wait(rows_b, gsb)
    s_start(c0 + 1, rows_b, ssb)
    s_wait(rows_a, ssa)
    s_wait(rows_b, ssb)


def _proj_kernel(x_ref, w_ref, bias_ref, out_ref):
    for j in range(_BL):
        out_ref[j] = (
            lax.dot_general(
                w_ref[...],
                x_ref[j],
                dimension_numbers=(((1,), (1,)), ((), ())),
                preferred_element_type=jnp.float32,
            )
            + bias_ref[...]
        )


def _proj_update_kernel(x_ref, w_ref, bias_ref, prev_ref, out_ref):
    del prev_ref
    _proj_kernel(x_ref, w_ref, bias_ref, out_ref)


def _onehot_proj_kernel(ids_ref, tt_ref, w_ref, bias_ref, out_ref):
    i = pl.program_id(0)
    v, bsz = out_ref.shape[1], out_ref.shape[2]
    for j in range(_BL):
        idrow = ids_ref[pl.ds(i * _BL + j, 1)]
        sel = jnp.where(
            lax.broadcasted_iota(jnp.int32, (v, bsz), 0) == idrow,
            jnp.float32(1),
            jnp.float32(0),
        )
        xt = jnp.dot(tt_ref[...], sel, preferred_element_type=jnp.float32)
        out_ref[j] = (
            jnp.dot(w_ref[...], xt, preferred_element_type=jnp.float32)
            + bias_ref[...]
        )


def kernel(input_ids, embed_table, proj_w, proj_b):
    b, l = input_ids.shape
    v, d = embed_table.shape
    n = b * l

    table_pad = jnp.pad(embed_table, ((0, 0), (0, _DPAD - d)))
    w_pad = jnp.pad(proj_w, ((0, 0), (0, _DPAD - d)))
    bias = proj_b.reshape(v, 1)
    ids = input_ids.T.reshape(n).astype(jnp.int32)

    def make_gather(n_c):
        return pl.kernel(
            _gather_body,
            out_type=jax.ShapeDtypeStruct((n_c, _DPAD), jnp.float32),
            mesh=plsc.VectorSubcoreMesh(
                core_axis_name="c", subcore_axis_name="s",
                num_cores=_NUM_CORES, num_subcores=_NUM_SUBCORES,
            ),
            scratch_types=[
                pltpu.VMEM_SHARED((v, _DPAD), jnp.float32),
                pltpu.VMEM((n_c // _NUM_WORKERS,), jnp.int32),
                pltpu.VMEM((_CHUNK, _DPAD), jnp.float32),
                pltpu.VMEM((_CHUNK, _DPAD), jnp.float32),
                pltpu.SemaphoreType.DMA,
                pltpu.SemaphoreType.DMA,
                pltpu.SemaphoreType.DMA,
                pltpu.SemaphoreType.DMA,
            ],
        )

    xs = [None]
    l_off = _SPLITS[0]
    for l_c in _SPLITS[1:]:
        n_c = l_c * b
        xs.append(
            make_gather(n_c)(
                table_pad, lax.slice(ids, (l_off * b,), ((l_off + l_c) * b,))
            ).reshape(l_c, b, _DPAD)
        )
        l_off += l_c

    common = dict(
        out_shape=jax.ShapeDtypeStruct((l, v, b), jnp.float32),
        compiler_params=pltpu.CompilerParams(
            dimension_semantics=("arbitrary",),
            vmem_limit_bytes=120 * 1024 * 1024,
        ),
    )
    x_spec = pl.BlockSpec((_BL, b, _DPAD), lambda i: (i, 0, 0))
    w_spec = pl.BlockSpec((v, _DPAD), lambda i: (0, 0))
    b_spec = pl.BlockSpec((v, 1), lambda i: (0, 0))

    ids0 = lax.slice(ids, (0,), (_SPLITS[0] * b,)).reshape(_SPLITS[0], b)
    tt_pad = jnp.pad(embed_table.T, ((0, _DPAD - d), (0, 0)))
    out_t = pl.pallas_call(
        _onehot_proj_kernel,
        grid=(_SPLITS[0] // _BL,),
        in_specs=[
            pl.BlockSpec((_SPLITS[0], b), lambda i: (0, 0)),
            pl.BlockSpec((_DPAD, v), lambda i: (0, 0)),
            w_spec,
            b_spec,
        ],
        out_specs=pl.BlockSpec((_BL, v, b), lambda i: (i, 0, 0)),
        **common,
    )(ids0, tt_pad, w_pad, bias)

    l_off = _SPLITS[0]
    for c in range(1, len(_SPLITS)):
        out_t = pl.pallas_call(
            _proj_update_kernel,
            grid=(_SPLITS[c] // _BL,),
            in_specs=[
                x_spec,
                w_spec,
                b_spec,
                pl.BlockSpec(memory_space=pl.ANY),
            ],
            out_specs=pl.BlockSpec(
                (_BL, v, b),
                functools.partial(lambda off, i: (off + i, 0, 0), l_off // _BL),
            ),
            input_output_aliases={3: 0},
            **common,
        )(xs[c], w_pad, bias, out_t)
        l_off += _SPLITS[c]

    return jnp.transpose(out_t, (2, 0, 1))


# speedup vs baseline: 1.0686x; 1.0096x over previous
import functools

import jax
import jax.numpy as jnp
from jax import lax
from jax.experimental import pallas as pl
from jax.experimental.pallas import tpu as pltpu
from jax.experimental.pallas import tpu_sc as plsc

_NUM_CORES = 2
_NUM_SUBCORES = 16
_NUM_WORKERS = _NUM_CORES * _NUM_SUBCORES

_DPAD = 128
_CHUNK = 64
_BL = 2
_SPLITS = (26, 24)


def _gather_body(table_hbm, ids_hbm, x_hbm, table_sh, idx_v, rows_a, rows_b, gsa, gsb, ssa, ssb):
    sid = lax.axis_index("s")
    wid = sid * _NUM_CORES + lax.axis_index("c")
    n = ids_hbm.shape[0]
    n_per_w = n // _NUM_WORKERS
    base = pl.multiple_of(wid * n_per_w, 8)

    @pl.when(sid == 0)
    def _():
        pltpu.sync_copy(table_hbm, table_sh)

    pltpu.sync_copy(ids_hbm.at[pl.ds(base, n_per_w)], idx_v)
    plsc.subcore_barrier()

    nch = n_per_w // _CHUNK
    nph = nch // 2

    def g_start(c, buf, sem):
        off = pl.multiple_of(c * _CHUNK, 8)
        pltpu.async_copy(table_sh.at[idx_v.at[pl.ds(off, _CHUNK)]], buf, sem)

    def g_wait(buf, sem):
        pltpu.make_async_copy(table_hbm.at[pl.ds(0, _CHUNK)], buf, sem).wait()

    def s_start(c, buf, sem):
        off = pl.multiple_of(base + c * _CHUNK, 8)
        pltpu.async_copy(buf, x_hbm.at[pl.ds(off, _CHUNK)], sem)

    def s_wait(buf, sem):
        pltpu.make_async_copy(buf, x_hbm.at[pl.ds(base, _CHUNK)], sem).wait()

    g_start(0, rows_a, gsa)
    g_wait(rows_a, gsa)
    s_start(0, rows_a, ssa)
    g_start(1, rows_b, gsb)
    g_wait(rows_b, gsb)
    s_start(1, rows_b, ssb)
    s_wait(rows_a, ssa)
    g_start(2, rows_a, gsa)

    def pair_step(i, carry):
        c0 = i * 2
        g_wait(rows_a, gsa)
        s_start(c0, rows_a, ssa)
        s_wait(rows_b, ssb)
        g_start(c0 + 1, rows_b, gsb)
        g_wait(rows_b, gsb)
        s_start(c0 + 1, rows_b, ssb)
        s_wait(rows_a, ssa)
        g_start(c0 + 2, rows_a, gsa)
        return carry

    lax.fori_loop(1, nph - 1, pair_step, 0)

    c0 = (nph - 1) * 2
    g_wait(rows_a, gsa)
    s_start(c0, rows_a, ssa)
    s_wait(rows_b, ssb)
    g_start(c0 + 1, rows_b, gsb)
    g_wait(rows_b, gsb)
    s_start(c0 + 1, rows_b, ssb)
    s_wait(rows_a, ssa)
    s_wait(rows_b, ssb)


def _proj_kernel(x_ref, w_ref, bias_ref, out_ref):
    for j in range(_BL):
        out_ref[j] = (
            lax.dot_general(
                w_ref[...],
                x_ref[j],
                dimension_numbers=(((1,), (1,)), ((), ())),
                preferred_element_type=jnp.float32,
            )
            + bias_ref[...]
        )


def _proj_update_kernel(x_ref, w_ref, bias_ref, prev_ref, out_ref):
    del prev_ref
    _proj_kernel(x_ref, w_ref, bias_ref, out_ref)


def _onehot_proj_kernel(ids_ref, tt_ref, w_ref, bias_ref, out_ref):
    i = pl.program_id(0)
    v, bsz = out_ref.shape[1], out_ref.shape[2]
    for j in range(_BL):
        idrow = ids_ref[pl.ds(i * _BL + j, 1)]
        sel = jnp.where(
            lax.broadcasted_iota(jnp.int32, (v, bsz), 0) == idrow,
            jnp.float32(1),
            jnp.float32(0),
        )
        xt = jnp.dot(tt_ref[...], sel, preferred_element_type=jnp.float32)
        out_ref[j] = (
            jnp.dot(w_ref[...], xt, preferred_element_type=jnp.float32)
            + bias_ref[...]
        )


def kernel(input_ids, embed_table, proj_w, proj_b):
    b, l = input_ids.shape
    v, d = embed_table.shape
    n = b * l

    table_pad = jnp.pad(embed_table, ((0, 0), (0, _DPAD - d)))
    w_pad = jnp.pad(proj_w, ((0, 0), (0, _DPAD - d)))
    bias = proj_b.reshape(v, 1)
    ids = input_ids.T.reshape(n).astype(jnp.int32)

    def make_gather(n_c):
        return pl.kernel(
            _gather_body,
            out_type=jax.ShapeDtypeStruct((n_c, _DPAD), jnp.float32),
            mesh=plsc.VectorSubcoreMesh(
                core_axis_name="c", subcore_axis_name="s",
                num_cores=_NUM_CORES, num_subcores=_NUM_SUBCORES,
            ),
            scratch_types=[
                pltpu.VMEM_SHARED((v, _DPAD), jnp.float32),
                pltpu.VMEM((n_c // _NUM_WORKERS,), jnp.int32),
                pltpu.VMEM((_CHUNK, _DPAD), jnp.float32),
                pltpu.VMEM((_CHUNK, _DPAD), jnp.float32),
                pltpu.SemaphoreType.DMA,
                pltpu.SemaphoreType.DMA,
                pltpu.SemaphoreType.DMA,
                pltpu.SemaphoreType.DMA,
            ],
        )

    xs = [None]
    l_off = _SPLITS[0]
    for l_c in _SPLITS[1:]:
        n_c = l_c * b
        xs.append(
            make_gather(n_c)(
                table_pad, lax.slice(ids, (l_off * b,), ((l_off + l_c) * b,))
            ).reshape(l_c, b, _DPAD)
        )
        l_off += l_c

    common = dict(
        out_shape=jax.ShapeDtypeStruct((l, v, b), jnp.float32),
        compiler_params=pltpu.CompilerParams(
            dimension_semantics=("arbitrary",),
            vmem_limit_bytes=120 * 1024 * 1024,
        ),
    )
    x_spec = pl.BlockSpec((_BL, b, _DPAD), lambda i: (i, 0, 0))
    w_spec = pl.BlockSpec((v, _DPAD), lambda i: (0, 0))
    b_spec = pl.BlockSpec((v, 1), lambda i: (0, 0))

    ids0 = lax.slice(ids, (0,), (_SPLITS[0] * b,)).reshape(_SPLITS[0], b)
    tt_pad = jnp.pad(embed_table.T, ((0, _DPAD - d), (0, 0)))
    out_t = pl.pallas_call(
        _onehot_proj_kernel,
        grid=(_SPLITS[0] // _BL,),
        in_specs=[
            pl.BlockSpec((_SPLITS[0], b), lambda i: (0, 0)),
            pl.BlockSpec((_DPAD, v), lambda i: (0, 0)),
            w_spec,
            b_spec,
        ],
        out_specs=pl.BlockSpec((_BL, v, b), lambda i: (i, 0, 0)),
        **common,
    )(ids0, tt_pad, w_pad, bias)

    l_off = _SPLITS[0]
    for c in range(1, len(_SPLITS)):
        out_t = pl.pallas_call(
            _proj_update_kernel,
            grid=(_SPLITS[c] // _BL,),
            in_specs=[
                x_spec,
                w_spec,
                b_spec,
                pl.BlockSpec(memory_space=pl.ANY),
            ],
            out_specs=pl.BlockSpec(
                (_BL, v, b),
                functools.partial(lambda off, i: (off + i, 0, 0), l_off // _BL),
            ),
            input_output_aliases={3: 0},
            **common,
        )(xs[c], w_pad, bias, out_t)
        l_off += _SPLITS[c]

    return jnp.transpose(out_t, (2, 0, 1))


# speedup vs baseline: 1.0773x; 1.0081x over previous
import functools

import jax
import jax.numpy as jnp
from jax import lax
from jax.experimental import pallas as pl
from jax.experimental.pallas import tpu as pltpu
from jax.experimental.pallas import tpu_sc as plsc

_NUM_CORES = 2
_NUM_SUBCORES = 16
_NUM_WORKERS = _NUM_CORES * _NUM_SUBCORES

_DPAD = 128
_CHUNK = 64
_BL = 2
_SPLITS = (30, 20)


def _gather_body(table_hbm, ids_hbm, x_hbm, table_sh, idx_v, rows_a, rows_b, gsa, gsb, ssa, ssb):
    sid = lax.axis_index("s")
    wid = sid * _NUM_CORES + lax.axis_index("c")
    n = ids_hbm.shape[0]
    n_per_w = n // _NUM_WORKERS
    base = pl.multiple_of(wid * n_per_w, 8)

    @pl.when(sid == 0)
    def _():
        pltpu.sync_copy(table_hbm, table_sh)

    pltpu.sync_copy(ids_hbm.at[pl.ds(base, n_per_w)], idx_v)
    plsc.subcore_barrier()

    nch = n_per_w // _CHUNK
    nph = nch // 2

    def g_start(c, buf, sem):
        off = pl.multiple_of(c * _CHUNK, 8)
        pltpu.async_copy(table_sh.at[idx_v.at[pl.ds(off, _CHUNK)]], buf, sem)

    def g_wait(buf, sem):
        pltpu.make_async_copy(table_hbm.at[pl.ds(0, _CHUNK)], buf, sem).wait()

    def s_start(c, buf, sem):
        off = pl.multiple_of(base + c * _CHUNK, 8)
        pltpu.async_copy(buf, x_hbm.at[pl.ds(off, _CHUNK)], sem)

    def s_wait(buf, sem):
        pltpu.make_async_copy(buf, x_hbm.at[pl.ds(base, _CHUNK)], sem).wait()

    g_start(0, rows_a, gsa)
    g_wait(rows_a, gsa)
    s_start(0, rows_a, ssa)
    g_start(1, rows_b, gsb)
    g_wait(rows_b, gsb)
    s_start(1, rows_b, ssb)
    s_wait(rows_a, ssa)
    g_start(2, rows_a, gsa)

    def pair_step(i, carry):
        c0 = i * 2
        g_wait(rows_a, gsa)
        s_start(c0, rows_a, ssa)
        s_wait(rows_b, ssb)
        g_start(c0 + 1, rows_b, gsb)
        g_wait(rows_b, gsb)
        s_start(c0 + 1, rows_b, ssb)
        s_wait(rows_a, ssa)
        g_start(c0 + 2, rows_a, gsa)
        return carry

    lax.fori_loop(1, nph - 1, pair_step, 0)

    c0 = (nph - 1) * 2
    g_wait(rows_a, gsa)
    s_start(c0, rows_a, ssa)
    s_wait(rows_b, ssb)
    g_start(c0 + 1, rows_b, gsb)
    g_wait(rows_b, gsb)
    s_start(c0 + 1, rows_b, ssb)
    s_wait(rows_a, ssa)
    s_wait(rows_b, ssb)


def _proj_kernel(x_ref, w_ref, bias_ref, out_ref):
    for j in range(_BL):
        out_ref[j] = (
            lax.dot_general(
                w_ref[...],
                x_ref[j],
                dimension_numbers=(((1,), (1,)), ((), ())),
                preferred_element_type=jnp.float32,
            )
            + bias_ref[...]
        )


def _proj_update_kernel(x_ref, w_ref, bias_ref, prev_ref, out_ref):
    del prev_ref
    _proj_kernel(x_ref, w_ref, bias_ref, out_ref)


def _onehot_proj_kernel(ids_ref, tt_ref, w_ref, bias_ref, out_ref):
    i = pl.program_id(0)
    v, bsz = out_ref.shape[1], out_ref.shape[2]
    for j in range(_BL):
        idrow = ids_ref[pl.ds(i * _BL + j, 1)]
        sel = jnp.where(
            lax.broadcasted_iota(jnp.int32, (v, bsz), 0) == idrow,
            jnp.float32(1),
            jnp.float32(0),
        )
        xt = jnp.dot(tt_ref[...], sel, preferred_element_type=jnp.float32)
        out_ref[j] = (
            jnp.dot(w_ref[...], xt, preferred_element_type=jnp.float32)
            + bias_ref[...]
        )


def kernel(input_ids, embed_table, proj_w, proj_b):
    b, l = input_ids.shape
    v, d = embed_table.shape
    n = b * l

    table_pad = jnp.pad(embed_table, ((0, 0), (0, _DPAD - d)))
    w_pad = jnp.pad(proj_w, ((0, 0), (0, _DPAD - d)))
    bias = proj_b.reshape(v, 1)
    ids = input_ids.T.reshape(n).astype(jnp.int32)

    def make_gather(n_c):
        return pl.kernel(
            _gather_body,
            out_type=jax.ShapeDtypeStruct((n_c, _DPAD), jnp.float32),
            mesh=plsc.VectorSubcoreMesh(
                core_axis_name="c", subcore_axis_name="s",
                num_cores=_NUM_CORES, num_subcores=_NUM_SUBCORES,
            ),
            scratch_types=[
                pltpu.VMEM_SHARED((v, _DPAD), jnp.float32),
                pltpu.VMEM((n_c // _NUM_WORKERS,), jnp.int32),
                pltpu.VMEM((_CHUNK, _DPAD), jnp.float32),
                pltpu.VMEM((_CHUNK, _DPAD), jnp.float32),
                pltpu.SemaphoreType.DMA,
                pltpu.SemaphoreType.DMA,
                pltpu.SemaphoreType.DMA,
                pltpu.SemaphoreType.DMA,
            ],
        )

    xs = [None]
    l_off = _SPLITS[0]
    for l_c in _SPLITS[1:]:
        n_c = l_c * b
        xs.append(
            make_gather(n_c)(
                table_pad, lax.slice(ids, (l_off * b,), ((l_off + l_c) * b,))
            ).reshape(l_c, b, _DPAD)
        )
        l_off += l_c

    common = dict(
        out_shape=jax.ShapeDtypeStruct((l, v, b), jnp.float32),
        compiler_params=pltpu.CompilerParams(
            dimension_semantics=("arbitrary",),
            vmem_limit_bytes=120 * 1024 * 1024,
        ),
    )
    x_spec = pl.BlockSpec((_BL, b, _DPAD), lambda i: (i, 0, 0))
    w_spec = pl.BlockSpec((v, _DPAD), lambda i: (0, 0))
    b_spec = pl.BlockSpec((v, 1), lambda i: (0, 0))

    ids0 = lax.slice(ids, (0,), (_SPLITS[0] * b,)).reshape(_SPLITS[0], b)
    tt_pad = jnp.pad(embed_table.T, ((0, _DPAD - d), (0, 0)))
    out_t = pl.pallas_call(
        _onehot_proj_kernel,
        grid=(_SPLITS[0] // _BL,),
        in_specs=[
            pl.BlockSpec((_SPLITS[0], b), lambda i: (0, 0)),
            pl.BlockSpec((_DPAD, v), lambda i: (0, 0)),
            w_spec,
            b_spec,
        ],
        out_specs=pl.BlockSpec((_BL, v, b), lambda i: (i, 0, 0)),
        **common,
    )(ids0, tt_pad, w_pad, bias)

    l_off = _SPLITS[0]
    for c in range(1, len(_SPLITS)):
        out_t = pl.pallas_call(
            _proj_update_kernel,
            grid=(_SPLITS[c] // _BL,),
            in_specs=[
                x_spec,
                w_spec,
                b_spec,
                pl.BlockSpec(memory_space=pl.ANY),
            ],
            out_specs=pl.BlockSpec(
                (_BL, v, b),
                functools.partial(lambda off, i: (off + i, 0, 0), l_off // _BL),
            ),
            input_output_aliases={3: 0},
            **common,
        )(xs[c], w_pad, bias, out_t)
        l_off += _SPLITS[c]

    return jnp.transpose(out_t, (2, 0, 1))
